# baseline (speedup 1.0000x reference)
import functools

import jax
import jax.numpy as jnp
from jax.experimental import pallas as pl
from jax.experimental.pallas import tpu as pltpu

_BIG = 1e9


def _min2_tree(u):
    w = u.shape[1]
    h = w // 2
    a, b = u[:, :h], u[:, h:]
    m1 = jnp.minimum(a, b)
    m2 = jnp.maximum(a, b)
    w = h
    while w > 1:
        h = w // 2
        a1, b1 = m1[:, :h], m1[:, h:]
        a2, b2 = m2[:, :h], m2[:, h:]
        m1, m2 = (
            jnp.minimum(a1, b1),
            jnp.minimum(jnp.maximum(a1, b1), jnp.minimum(a2, b2)),
        )
        w = h
    return m1, m2


def _knn_loss_block(x_row_ref, xt_ref, pen_ref, out_ref,
                    xt_aug_ref, d2_ref, *, k, rows_per_blk):
    i = pl.program_id(0)
    dim = xt_ref.shape[0]

    @pl.when(i == 0)
    def _init_xt_aug():
        xt = xt_ref[:]
        xt_aug_ref[0:dim, :] = xt
        xt_aug_ref[dim:dim + 1, :] = jnp.sum(xt * xt, axis=0, keepdims=True)
        xt_aug_ref[dim + 1:dim + 2, :] = jnp.ones_like(xt_aug_ref[dim + 1:dim + 2, :])

    xr = x_row_ref[:]
    sq_r = jnp.sum(xr * xr, axis=1, keepdims=True)
    xr_aug = jnp.concatenate(
        [xr * -2.0, jnp.ones_like(sq_r), sq_r], axis=1)
    d2_ref[:, :] = jax.lax.dot_general(
        xr_aug, xt_aug_ref[:], (((1,), (0,)), ((), ())),
        preferred_element_type=jnp.float32,
    )
    d2_ref[:, pl.ds(i * rows_per_blk, rows_per_blk)] += pen_ref[:]
    u = jnp.maximum(d2_ref[:, :], 0.0)

    m1q, m2q = _min2_tree(u)
    s = jnp.exp(-jnp.sqrt(u))
    denom = jnp.sum(s, axis=1, keepdims=True)

    loss_rows = (jnp.sqrt(m1q) + jnp.sqrt(m2q)) * (1.0 / k) + jnp.log(denom)
    part = jnp.sum(loss_rows)[None, None]

    @pl.when(i == 0)
    def _init_out():
        out_ref[:, :] = jnp.zeros((1, 1), jnp.float32)

    out_ref[:, :] += part


def kernel(x):
    n, d = x.shape
    rows_per_blk = 512
    pen = _BIG * jnp.eye(rows_per_blk, dtype=jnp.float32)
    out = pl.pallas_call(
        functools.partial(_knn_loss_block, k=2, rows_per_blk=rows_per_blk),
        grid=(n // rows_per_blk,),
        in_specs=[
            pl.BlockSpec((rows_per_blk, d), lambda i: (i, 0)),
            pl.BlockSpec((d, n), lambda i: (0, 0)),
            pl.BlockSpec((rows_per_blk, rows_per_blk), lambda i: (0, 0)),
        ],
        out_specs=pl.BlockSpec((1, 1), lambda i: (0, 0)),
        out_shape=jax.ShapeDtypeStruct((1, 1), jnp.float32),
        scratch_shapes=[
            pltpu.VMEM((d + 2, n), jnp.float32),
            pltpu.VMEM((rows_per_blk, n), jnp.float32),
        ],
    )(x, x.T, pen)
    return out[0, 0] / n
